# baseline (speedup 1.0000x reference)
import jax
import jax.numpy as jnp
from jax.experimental import pallas as pl
from jax.experimental.pallas import tpu as pltpu

_T = 20
_N = 10000
_D = 128
_R = _T * _N

_BR = 2000
_GRID = _R // _BR


def _tc_body(obs_ref, nd_ref, pos_ref, neg_ref, out_ref):
    obs = obs_ref[...]
    a = (obs == 1).astype(jnp.float32)
    b = (obs == -1).astype(jnp.float32)
    out_ref[...] = nd_ref[...] + a * pos_ref[...] + b * neg_ref[...]


def kernel(node_data, edge_weights, pos_test_spike, neg_test_spike, observations):
    nd = node_data.reshape(_R, _D)
    obs = observations.reshape(_R, 1).astype(jnp.int32)
    pos = pos_test_spike.reshape(1, _D)
    neg = neg_test_spike.reshape(1, _D)

    out = pl.pallas_call(
        _tc_body,
        grid=(_GRID,),
        in_specs=[
            pl.BlockSpec((_BR, 1), lambda i: (i, 0)),
            pl.BlockSpec((_BR, _D), lambda i: (i, 0)),
            pl.BlockSpec((1, _D), lambda i: (0, 0)),
            pl.BlockSpec((1, _D), lambda i: (0, 0)),
        ],
        out_specs=pl.BlockSpec((_BR, _D), lambda i: (i, 0)),
        out_shape=jax.ShapeDtypeStruct((_R, _D), jnp.float32),
        compiler_params=pltpu.CompilerParams(
            dimension_semantics=("arbitrary",),
        ),
    )(obs, nd, pos, neg)

    return out.reshape(_T, _N, _D), edge_weights


# speedup vs baseline: 1.8152x; 1.8152x over previous
import jax
import jax.numpy as jnp
from jax.experimental import pallas as pl
from jax.experimental.pallas import tpu as pltpu

_T = 20
_N = 10000
_D = 128
_R = _T * _N

_BR = 2000
_GRID = _R // _BR


def _tc_body(obs_ref, nd_ref, spikes_ref, out_ref):
    obs = obs_ref[0]
    a = (obs == 1).astype(jnp.float32)
    b = (obs == -1).astype(jnp.float32)
    coef = jnp.concatenate([a, b], axis=0)
    contrib = jax.lax.dot_general(
        coef, spikes_ref[...],
        dimension_numbers=(((0,), (0,)), ((), ())),
        preferred_element_type=jnp.float32,
    )
    out_ref[...] = nd_ref[...] + contrib


def kernel(node_data, edge_weights, pos_test_spike, neg_test_spike, observations):
    nd = node_data.reshape(_R, _D)
    obs = observations.reshape(_GRID, 1, _BR).astype(jnp.int32)
    spikes = jnp.stack([pos_test_spike, neg_test_spike], axis=0)

    out = pl.pallas_call(
        _tc_body,
        grid=(_GRID,),
        in_specs=[
            pl.BlockSpec((1, 1, _BR), lambda i: (i, 0, 0)),
            pl.BlockSpec((_BR, _D), lambda i: (i, 0)),
            pl.BlockSpec((2, _D), lambda i: (0, 0)),
        ],
        out_specs=pl.BlockSpec((_BR, _D), lambda i: (i, 0)),
        out_shape=jax.ShapeDtypeStruct((_R, _D), jnp.float32),
        compiler_params=pltpu.CompilerParams(
            dimension_semantics=("arbitrary",),
        ),
    )(obs, nd, spikes)

    return out.reshape(_T, _N, _D), edge_weights


# speedup vs baseline: 2.5071x; 1.3812x over previous
import jax
import jax.numpy as jnp
from jax.experimental import pallas as pl
from jax.experimental.pallas import tpu as pltpu

_T = 20
_N = 10000
_D = 128
_R = _T * _N

_BR = 4000
_GRID = _R // _BR


def _tc_body(obs_ref, nd_ref, spikes_ref, out_ref):
    obs = obs_ref[0]
    a = (obs == 1).astype(jnp.float32)
    b = (obs == -1).astype(jnp.float32)
    coef = jnp.concatenate([a, b], axis=0)
    contrib = jax.lax.dot_general(
        coef, spikes_ref[...],
        dimension_numbers=(((0,), (0,)), ((), ())),
        preferred_element_type=jnp.float32,
    )
    out_ref[...] = nd_ref[...] + contrib


def kernel(node_data, edge_weights, pos_test_spike, neg_test_spike, observations):
    nd = node_data.reshape(_R, _D)
    obs = observations.reshape(_GRID, 1, _BR).astype(jnp.int32)
    spikes = jnp.stack([pos_test_spike, neg_test_spike], axis=0)

    out = pl.pallas_call(
        _tc_body,
        grid=(_GRID,),
        in_specs=[
            pl.BlockSpec((1, 1, _BR), lambda i: (i, 0, 0)),
            pl.BlockSpec((_BR, _D), lambda i: (i, 0)),
            pl.BlockSpec((2, _D), lambda i: (0, 0)),
        ],
        out_specs=pl.BlockSpec((_BR, _D), lambda i: (i, 0)),
        out_shape=jax.ShapeDtypeStruct((_R, _D), jnp.float32),
        compiler_params=pltpu.CompilerParams(
            dimension_semantics=("arbitrary",),
        ),
    )(obs, nd, spikes)

    return out.reshape(_T, _N, _D), edge_weights


# speedup vs baseline: 2.9574x; 1.1796x over previous
import jax
import jax.numpy as jnp
from jax.experimental import pallas as pl
from jax.experimental.pallas import tpu as pltpu

_T = 20
_N = 10000
_D = 128
_R = _T * _N

_BR = 8000
_GRID = _R // _BR


def _tc_body(obs_ref, nd_ref, spikes_ref, out_ref):
    obs = obs_ref[0]
    a = (obs == 1).astype(jnp.float32)
    b = (obs == -1).astype(jnp.float32)
    coef = jnp.concatenate([a, b], axis=0)
    contrib = jax.lax.dot_general(
        coef, spikes_ref[...],
        dimension_numbers=(((0,), (0,)), ((), ())),
        preferred_element_type=jnp.float32,
    )
    out_ref[...] = nd_ref[...] + contrib


def kernel(node_data, edge_weights, pos_test_spike, neg_test_spike, observations):
    nd = node_data.reshape(_R, _D)
    obs = observations.reshape(_GRID, 1, _BR).astype(jnp.int32)
    spikes = jnp.stack([pos_test_spike, neg_test_spike], axis=0)

    out = pl.pallas_call(
        _tc_body,
        grid=(_GRID,),
        in_specs=[
            pl.BlockSpec((1, 1, _BR), lambda i: (i, 0, 0)),
            pl.BlockSpec((_BR, _D), lambda i: (i, 0)),
            pl.BlockSpec((2, _D), lambda i: (0, 0)),
        ],
        out_specs=pl.BlockSpec((_BR, _D), lambda i: (i, 0)),
        out_shape=jax.ShapeDtypeStruct((_R, _D), jnp.float32),
        compiler_params=pltpu.CompilerParams(
            dimension_semantics=("arbitrary",),
        ),
    )(obs, nd, spikes)

    return out.reshape(_T, _N, _D), edge_weights


# speedup vs baseline: 3.0819x; 1.0421x over previous
import jax
import jax.numpy as jnp
from jax.experimental import pallas as pl
from jax.experimental.pallas import tpu as pltpu

_T = 20
_N = 10000
_D = 128
_R = _T * _N

_BR = 20000
_GRID = _R // _BR


def _tc_body(obs_ref, nd_ref, spikes_ref, out_ref):
    obs = obs_ref[0]
    a = (obs == 1).astype(jnp.float32)
    b = (obs == -1).astype(jnp.float32)
    coef = jnp.concatenate([a, b], axis=0)
    contrib = jax.lax.dot_general(
        coef, spikes_ref[...],
        dimension_numbers=(((0,), (0,)), ((), ())),
        preferred_element_type=jnp.float32,
    )
    out_ref[...] = nd_ref[...] + contrib


def kernel(node_data, edge_weights, pos_test_spike, neg_test_spike, observations):
    nd = node_data.reshape(_R, _D)
    obs = observations.reshape(_GRID, 1, _BR).astype(jnp.int32)
    spikes = jnp.stack([pos_test_spike, neg_test_spike], axis=0)

    out = pl.pallas_call(
        _tc_body,
        grid=(_GRID,),
        in_specs=[
            pl.BlockSpec((1, 1, _BR), lambda i: (i, 0, 0)),
            pl.BlockSpec((_BR, _D), lambda i: (i, 0)),
            pl.BlockSpec((2, _D), lambda i: (0, 0)),
        ],
        out_specs=pl.BlockSpec((_BR, _D), lambda i: (i, 0)),
        out_shape=jax.ShapeDtypeStruct((_R, _D), jnp.float32),
        compiler_params=pltpu.CompilerParams(
            dimension_semantics=("arbitrary",),
        ),
    )(obs, nd, spikes)

    return out.reshape(_T, _N, _D), edge_weights
